# baseline (speedup 1.0000x reference)
import functools

import jax
import jax.numpy as jnp
from jax.experimental import pallas as pl
from jax.experimental.pallas import tpu as pltpu


def _moe_body(n_f, blk_f, h_ref, gw_ref, gb_ref, w1_ref, b1_ref, w3_ref,
              b3_ref, w2_ref, b2_ref, out_ref, logits_ref, act_ref):
    f = pl.program_id(1)
    dn = (((1,), (1,)), ((), ()))
    h = h_ref[...].astype(jnp.bfloat16)
    a1 = jax.lax.dot_general(h, w1_ref[...], dn,
                             preferred_element_type=jnp.float32) + b1_ref[...]
    a3 = jax.lax.dot_general(h, w3_ref[...], dn,
                             preferred_element_type=jnp.float32) + b3_ref[...]
    act = (a1 * jax.nn.sigmoid(a1)) * a3
    act_ref[:, pl.ds(f * blk_f, blk_f)] = act.astype(jnp.bfloat16)

    @pl.when(f == 0)
    def _gate():
        logits_ref[...] = jax.lax.dot_general(
            h, gw_ref[...], dn, preferred_element_type=jnp.float32
        ) + gb_ref[...]

    @pl.when(f == n_f - 1)
    def _down():
        out_ref[...] = jax.lax.dot_general(
            act_ref[...], w2_ref[...], dn,
            preferred_element_type=jnp.float32) + b2_ref[...]


def _fused_moe(h, gate_w, gate_b, w1_w, w1_b, w3_w, w3_b, w2_w, w2_b,
               blk_t, blk_f):
    n_tokens, hidden = h.shape
    ffn = w1_w.shape[0]
    n_experts = gate_w.shape[0]
    n_t = n_tokens // blk_t
    n_f = ffn // blk_f

    grid = (n_t, n_f)
    out, logits = pl.pallas_call(
        functools.partial(_moe_body, n_f, blk_f),
        grid=grid,
        in_specs=[
            pl.BlockSpec((blk_t, hidden), lambda t, f: (t, 0)),
            pl.BlockSpec((n_experts, hidden), lambda t, f: (0, 0)),
            pl.BlockSpec((1, n_experts), lambda t, f: (0, 0)),
            pl.BlockSpec((blk_f, hidden), lambda t, f: (f, 0)),
            pl.BlockSpec((1, blk_f), lambda t, f: (0, f)),
            pl.BlockSpec((blk_f, hidden), lambda t, f: (f, 0)),
            pl.BlockSpec((1, blk_f), lambda t, f: (0, f)),
            pl.BlockSpec((hidden, ffn), lambda t, f: (0, 0)),
            pl.BlockSpec((1, hidden), lambda t, f: (0, 0)),
        ],
        out_specs=[
            pl.BlockSpec((blk_t, hidden), lambda t, f: (t, 0)),
            pl.BlockSpec((blk_t, n_experts), lambda t, f: (t, 0)),
        ],
        out_shape=[
            jax.ShapeDtypeStruct((n_tokens, hidden), jnp.float32),
            jax.ShapeDtypeStruct((n_tokens, n_experts), jnp.float32),
        ],
        scratch_shapes=[pltpu.VMEM((blk_t, ffn), jnp.bfloat16)],
        compiler_params=pltpu.CompilerParams(
            dimension_semantics=("parallel", "arbitrary")),
    )(h, gate_w, gate_b, w1_w, w1_b, w3_w, w3_b, w2_w, w2_b)
    return out, logits


def kernel(hidden_states, gate_w, gate_b, w1_w, w1_b, w3_w, w3_b, w2_w, w2_b):
    batch, seq, hidden = hidden_states.shape
    h = hidden_states.reshape(batch * seq, hidden)
    out, logits = _fused_moe(
        h, gate_w.astype(jnp.bfloat16), gate_b.reshape(1, -1),
        w1_w.astype(jnp.bfloat16), w1_b.reshape(1, -1),
        w3_w.astype(jnp.bfloat16), w3_b.reshape(1, -1),
        w2_w.astype(jnp.bfloat16), w2_b.reshape(1, -1),
        blk_t=256, blk_f=512,
    )
    return out.reshape(batch, seq, hidden), logits


# speedup vs baseline: 1.3384x; 1.3384x over previous
import functools

import jax
import jax.numpy as jnp
from jax.experimental import pallas as pl
from jax.experimental.pallas import tpu as pltpu


def _moe_body(n_f, blk_f, group, h_ref, gw_ref, gb_ref, w1_ref, b1_ref,
              w3_ref, b3_ref, w2_ref, b2_ref, out_ref, logits_ref, act_ref):
    f = pl.program_id(1)
    dn = (((1,), (1,)), ((), ()))
    h = h_ref[...].astype(jnp.bfloat16)
    a1 = jax.lax.dot_general(h, w1_ref[...], dn,
                             preferred_element_type=jnp.float32) + b1_ref[...]
    a3 = jax.lax.dot_general(h, w3_ref[...], dn,
                             preferred_element_type=jnp.float32) + b3_ref[...]
    act = (a1 * jax.nn.sigmoid(a1)) * a3
    col = jax.lax.rem(f, group) * blk_f
    act_ref[:, pl.ds(col, blk_f)] = act.astype(jnp.bfloat16)

    @pl.when(f == 0)
    def _gate():
        logits_ref[...] = jax.lax.dot_general(
            h, gw_ref[...], dn, preferred_element_type=jnp.float32
        ) + gb_ref[...]

    @pl.when(f == group - 1)
    def _down_first():
        out_ref[...] = jax.lax.dot_general(
            act_ref[...], w2_ref[...], dn,
            preferred_element_type=jnp.float32) + b2_ref[...]

    @pl.when(jnp.logical_and(f > group - 1, jax.lax.rem(f, group) == group - 1))
    def _down_accum():
        out_ref[...] += jax.lax.dot_general(
            act_ref[...], w2_ref[...], dn,
            preferred_element_type=jnp.float32)


def _fused_moe(h, gate_w, gate_b, w1_w, w1_b, w3_w, w3_b, w2_w, w2_b,
               blk_t, blk_f, group=2):
    n_tokens, hidden = h.shape
    ffn = w1_w.shape[0]
    n_experts = gate_w.shape[0]
    n_t = n_tokens // blk_t
    n_f = ffn // blk_f
    assert n_f % group == 0 and n_f >= 2 * group

    grid = (n_t, n_f)
    out, logits = pl.pallas_call(
        functools.partial(_moe_body, n_f, blk_f, group),
        grid=grid,
        in_specs=[
            pl.BlockSpec((blk_t, hidden), lambda t, f: (t, 0)),
            pl.BlockSpec((n_experts, hidden), lambda t, f: (0, 0)),
            pl.BlockSpec((1, n_experts), lambda t, f: (0, 0)),
            pl.BlockSpec((blk_f, hidden), lambda t, f: (f, 0)),
            pl.BlockSpec((1, blk_f), lambda t, f: (0, f)),
            pl.BlockSpec((blk_f, hidden), lambda t, f: (f, 0)),
            pl.BlockSpec((1, blk_f), lambda t, f: (0, f)),
            pl.BlockSpec((hidden, group * blk_f),
                         lambda t, f: (0, f // group)),
            pl.BlockSpec((1, hidden), lambda t, f: (0, 0)),
        ],
        out_specs=[
            pl.BlockSpec((blk_t, hidden), lambda t, f: (t, 0)),
            pl.BlockSpec((blk_t, n_experts), lambda t, f: (t, 0)),
        ],
        out_shape=[
            jax.ShapeDtypeStruct((n_tokens, hidden), jnp.float32),
            jax.ShapeDtypeStruct((n_tokens, n_experts), jnp.float32),
        ],
        scratch_shapes=[pltpu.VMEM((blk_t, group * blk_f), jnp.bfloat16)],
        compiler_params=pltpu.CompilerParams(
            dimension_semantics=("parallel", "arbitrary")),
    )(h, gate_w, gate_b, w1_w, w1_b, w3_w, w3_b, w2_w, w2_b)
    return out, logits


def kernel(hidden_states, gate_w, gate_b, w1_w, w1_b, w3_w, w3_b, w2_w, w2_b):
    batch, seq, hidden = hidden_states.shape
    h = hidden_states.reshape(batch * seq, hidden)
    out, logits = _fused_moe(
        h, gate_w.astype(jnp.bfloat16), gate_b.reshape(1, -1),
        w1_w.astype(jnp.bfloat16), w1_b.reshape(1, -1),
        w3_w.astype(jnp.bfloat16), w3_b.reshape(1, -1),
        w2_w.astype(jnp.bfloat16), w2_b.reshape(1, -1),
        blk_t=512, blk_f=1024, group=2,
    )
    return out.reshape(batch, seq, hidden), logits
